# baseline (speedup 1.0000x reference)
import functools

import jax
import jax.numpy as jnp
from jax import lax
from jax.experimental import pallas as pl
from jax.experimental.pallas import tpu as pltpu
from jax.experimental.pallas import tpu_sc as plsc

_CH = 128
_CW = 16


def _adapter_block(x_ref, nt_ref, w_ref, b_ref, o_ref):
    x = x_ref[...]
    nt = nt_ref[...]
    acc = jnp.zeros_like(x)
    for t in range(w_ref.shape[0]):
        y = jnp.tanh(jnp.dot(x, w_ref[t], preferred_element_type=jnp.float32)
                     + b_ref[t])
        acc = jnp.where(nt == t, y, acc)
    o_ref[...] = acc


def _layer_block(s0_ref, s1_ref, c_ref, h_ref, rel_ref, w_ref, b_ref,
                 o_ref):
    cnt = c_ref[...]
    deg = jnp.maximum(jnp.sum(cnt, axis=1, keepdims=True), 1.0)
    base = (s0_ref[...] + s1_ref[...]
            + jnp.dot(cnt, rel_ref[...], preferred_element_type=jnp.float32))
    agg = base / deg
    o_ref[...] = (jnp.tanh(jnp.dot(agg, w_ref[...],
                                   preferred_element_type=jnp.float32)
                           + b_ref[...])
                  + h_ref[...])


def _make_sc_scatter(np_rows, width, nch0, nch1, rpt, nc, ns):
    mesh = plsc.VectorSubcoreMesh(core_axis_name="c", subcore_axis_name="s")

    @functools.partial(
        pl.kernel,
        out_type=jax.ShapeDtypeStruct((nc * np_rows, width), jnp.float32),
        mesh=mesh,
        scratch_types=[
            pltpu.VMEM((2, _CH), jnp.int32),
            pltpu.VMEM((_CH, width), jnp.float32),
            pltpu.SemaphoreType.DMA,
            pltpu.VMEM_SHARED((np_rows, width), jnp.float32),
        ],
    )
    def seg(table_hbm, pair_hbm, zeros_hbm, out_hbm, idx, rows, sem, acc):
        c = lax.axis_index("c")
        s = lax.axis_index("s")
        r0 = s * rpt
        nch_c = jnp.where(c == 0, nch0, nch1)
        base = jnp.where(c == 0, s * nch0, ns * nch0 + s * nch1)
        pltpu.sync_copy(zeros_hbm.at[pl.ds(r0, rpt)], acc.at[pl.ds(r0, rpt)])
        plsc.subcore_barrier()

        def body(j, carry):
            pltpu.sync_copy(pair_hbm.at[base + j], idx)
            pltpu.async_copy(table_hbm.at[idx.at[0]], rows, sem).wait()
            pltpu.sync_copy(rows, acc.at[idx.at[1]], add=True)
            return carry

        lax.fori_loop(0, nch_c, body, 0)
        plsc.subcore_barrier()
        pltpu.sync_copy(acc.at[pl.ds(r0, rpt)],
                        out_hbm.at[pl.ds(c * np_rows + r0, rpt)])

    return seg


def _make_sc_counts(np_rows, nr, nch, nc, ns):
    mesh = plsc.VectorSubcoreMesh(core_axis_name="c", subcore_axis_name="s")
    nw = nc * ns
    flat = np_rows * nr

    @functools.partial(
        pl.kernel,
        out_type=jax.ShapeDtypeStruct((nw, flat), jnp.float32),
        mesh=mesh,
        scratch_types=[
            pltpu.VMEM((nch, _CH), jnp.int32),
            pltpu.VMEM((nch, _CH), jnp.int32),
            pltpu.VMEM((flat,), jnp.float32),
        ],
        compiler_params=pltpu.CompilerParams(needs_layout_passes=False),
    )
    def counts(et_hbm, dst_hbm, out_hbm, ei, di, acc):
        c = lax.axis_index("c")
        s = lax.axis_index("s")
        w = c * ns + s
        zeros16 = jnp.zeros((16,), jnp.float32)
        ones = jnp.ones((16,), jnp.float32)
        pltpu.sync_copy(et_hbm.at[w], ei)
        pltpu.sync_copy(dst_hbm.at[w], di)

        def zbody(i, carry):
            acc[pl.ds(i * 16, 16)] = zeros16
            return carry

        lax.fori_loop(0, flat // 16, zbody, 0)

        def body(i, carry):
            for g in range(_CH // 16):
                idx = di[i, pl.ds(g * 16, 16)] * nr + ei[i, pl.ds(g * 16, 16)]
                plsc.addupdate_scatter(acc, [idx], ones)
            return carry

        lax.fori_loop(0, nch, body, 0)
        pltpu.sync_copy(acc, out_hbm.at[w])

    return counts


def _reduce_counts_block(parts_ref, o_ref):
    o_ref[...] = jnp.sum(parts_ref[...], axis=0)


def kernel(node_feature, node_type, edge_index, edge_type, node_position,
           adapt_W, adapt_b, conv_W, conv_b, rel_emb):
    n, d = node_feature.shape
    num_types = adapt_W.shape[0]
    num_layers, num_rel, _ = rel_emb.shape
    e = edge_index.shape[1]

    info = plsc.get_sparse_core_info()
    nc, ns = info.num_cores, info.num_subcores
    nw = nc * ns

    np_rows = ((n + 1 + 8 * ns - 1) // (8 * ns)) * (8 * ns)
    rpt = np_rows // ns
    nch = -(-e // (nw * _CH))
    epw = nch * _CH
    e_pad = epw * nw
    pad_e = e_pad - e

    src = jnp.concatenate(
        [edge_index[0], jnp.zeros((pad_e,), jnp.int32)])
    dst = jnp.concatenate(
        [edge_index[1], jnp.full((pad_e,), n, jnp.int32)])
    et = jnp.concatenate(
        [edge_type, jnp.zeros((pad_e,), jnp.int32)])

    x_p = jnp.zeros((np_rows, d), jnp.float32).at[:n].set(node_feature)
    nt_p = jnp.full((np_rows, 1), num_types, jnp.int32).at[:n, 0].set(node_type)
    zeros_nd = jnp.zeros((np_rows, d), jnp.float32)

    blk = np_rows // 8
    grid = (np_rows // blk,)
    full3 = lambda shape: pl.BlockSpec(shape, lambda i: (0, 0, 0))
    full2 = lambda shape: pl.BlockSpec(shape, lambda i: (0, 0))
    row_blk = lambda w: pl.BlockSpec((blk, w), lambda i: (i, 0))

    adapter = pl.pallas_call(
        _adapter_block,
        grid=grid,
        in_specs=[row_blk(d), row_blk(1),
                  full3((num_types, d, d)), full3((num_types, 1, d))],
        out_specs=row_blk(d),
        out_shape=jax.ShapeDtypeStruct((np_rows, d), jnp.float32),
    )
    h = adapter(x_p, nt_p, adapt_W, adapt_b[:, None, :])

    sc_counts = _make_sc_counts(np_rows, num_rel, nch, nc, ns)
    cnt_parts = sc_counts(et.reshape(nw, nch, _CH),
                          dst.reshape(nw, nch, _CH)).reshape(
                              nw, np_rows, num_rel)

    reduce_counts = pl.pallas_call(
        _reduce_counts_block,
        grid=grid,
        in_specs=[pl.BlockSpec((nw, blk, num_rel), lambda i: (0, i, 0))],
        out_specs=pl.BlockSpec((blk, num_rel), lambda i: (i, 0)),
        out_shape=jax.ShapeDtypeStruct((np_rows, num_rel), jnp.float32),
    )
    cmat = reduce_counts(cnt_parts)

    pairs = jnp.stack([src.reshape(nw * nch, _CH),
                       dst.reshape(nw * nch, _CH)], axis=1)
    nch1 = (2 * nch * 61 + 50) // 100
    nch0 = 2 * nch - nch1
    sc_seg = _make_sc_scatter(np_rows, d, nch0, nch1, rpt, nc, ns)

    layer = pl.pallas_call(
        _layer_block,
        grid=grid,
        in_specs=[row_blk(d), row_blk(d), row_blk(num_rel),
                  row_blk(d), full2((num_rel, d)), full2((d, d)),
                  full2((1, d))],
        out_specs=row_blk(d),
        out_shape=jax.ShapeDtypeStruct((np_rows, d), jnp.float32),
    )

    for l in range(num_layers):
        part = sc_seg(h, pairs, zeros_nd)
        h = layer(part[:np_rows], part[np_rows:], cmat, h,
                  rel_emb[l], conv_W[l], conv_b[l][None, :])

    return h[:n]


# speedup vs baseline: 1.1426x; 1.1426x over previous
import functools

import jax
import jax.numpy as jnp
from jax import lax
from jax.experimental import pallas as pl
from jax.experimental.pallas import tpu as pltpu
from jax.experimental.pallas import tpu_sc as plsc

_CH = 128
_CW = 16


def _adapter_block(x_ref, nt_ref, w_ref, b_ref, o_ref):
    x = x_ref[...]
    nt = nt_ref[...]
    acc = jnp.zeros_like(x)
    for t in range(w_ref.shape[0]):
        y = jnp.tanh(jnp.dot(x, w_ref[t], preferred_element_type=jnp.float32)
                     + b_ref[t])
        acc = jnp.where(nt == t, y, acc)
    o_ref[...] = acc


def _layer_block(s0_ref, s1_ref, c_ref, h_ref, rel_ref, w_ref, b_ref,
                 o_ref):
    cnt = c_ref[...]
    deg = jnp.maximum(jnp.sum(cnt, axis=1, keepdims=True), 1.0)
    base = (s0_ref[...] + s1_ref[...]
            + jnp.dot(cnt, rel_ref[...], preferred_element_type=jnp.float32))
    agg = base / deg
    o_ref[...] = (jnp.tanh(jnp.dot(agg, w_ref[...],
                                   preferred_element_type=jnp.float32)
                           + b_ref[...])
                  + h_ref[...])


def _make_sc_scatter(np_rows, width, nch0, nch1, rpt, nc, ns):
    mesh = plsc.VectorSubcoreMesh(core_axis_name="c", subcore_axis_name="s")

    @functools.partial(
        pl.kernel,
        out_type=jax.ShapeDtypeStruct((nc * np_rows, width), jnp.float32),
        mesh=mesh,
        scratch_types=[
            pltpu.VMEM((2, _CH), jnp.int32),
            pltpu.VMEM((_CH, width), jnp.float32),
            pltpu.SemaphoreType.DMA,
            pltpu.VMEM_SHARED((np_rows, width), jnp.float32),
        ],
    )
    def seg(table_hbm, pair_hbm, zeros_hbm, out_hbm, idx, rows, sem, acc):
        c = lax.axis_index("c")
        s = lax.axis_index("s")
        r0 = s * rpt
        nch_c = jnp.where(c == 0, nch0, nch1)
        base = jnp.where(c == 0, s * nch0, ns * nch0 + s * nch1)
        pltpu.sync_copy(zeros_hbm.at[pl.ds(r0, rpt)], acc.at[pl.ds(r0, rpt)])
        plsc.subcore_barrier()

        def body(j, carry):
            pltpu.sync_copy(pair_hbm.at[base + j], idx)
            pltpu.async_copy(table_hbm.at[idx.at[0]], rows, sem).wait()
            pltpu.sync_copy(rows, acc.at[idx.at[1]], add=True)
            return carry

        lax.fori_loop(0, nch_c, body, 0)
        plsc.subcore_barrier()
        pltpu.sync_copy(acc.at[pl.ds(r0, rpt)],
                        out_hbm.at[pl.ds(c * np_rows + r0, rpt)])

    return seg


def _make_sc_counts(np_rows, nr, nch, nc, ns):
    mesh = plsc.VectorSubcoreMesh(core_axis_name="c", subcore_axis_name="s")
    nw = nc * ns
    flat = np_rows * nr

    @functools.partial(
        pl.kernel,
        out_type=jax.ShapeDtypeStruct((nw, flat), jnp.float32),
        mesh=mesh,
        scratch_types=[
            pltpu.VMEM((nch, _CH), jnp.int32),
            pltpu.VMEM((nch, _CH), jnp.int32),
            pltpu.VMEM((flat,), jnp.float32),
        ],
        compiler_params=pltpu.CompilerParams(needs_layout_passes=False),
    )
    def counts(et_hbm, dst_hbm, out_hbm, ei, di, acc):
        c = lax.axis_index("c")
        s = lax.axis_index("s")
        w = c * ns + s
        zeros16 = jnp.zeros((16,), jnp.float32)
        ones = jnp.ones((16,), jnp.float32)
        pltpu.sync_copy(et_hbm.at[w], ei)
        pltpu.sync_copy(dst_hbm.at[w], di)

        def zbody(i, carry):
            acc[pl.ds(i * 16, 16)] = zeros16
            return carry

        lax.fori_loop(0, flat // 16, zbody, 0)

        def body(i, carry):
            for g in range(_CH // 16):
                idx = di[i, pl.ds(g * 16, 16)] * nr + ei[i, pl.ds(g * 16, 16)]
                plsc.addupdate_scatter(acc, [idx], ones)
            return carry

        lax.fori_loop(0, nch, body, 0)
        pltpu.sync_copy(acc, out_hbm.at[w])

    return counts


def _reduce_counts_block(parts_ref, o_ref):
    o_ref[...] = jnp.sum(parts_ref[...], axis=0)


def kernel(node_feature, node_type, edge_index, edge_type, node_position,
           adapt_W, adapt_b, conv_W, conv_b, rel_emb):
    n, d = node_feature.shape
    num_types = adapt_W.shape[0]
    num_layers, num_rel, _ = rel_emb.shape
    e = edge_index.shape[1]

    info = plsc.get_sparse_core_info()
    nc, ns = info.num_cores, info.num_subcores
    nw = nc * ns

    np_rows = ((n + 1 + 8 * ns - 1) // (8 * ns)) * (8 * ns)
    rpt = np_rows // ns
    nch = -(-e // (nw * _CH))
    epw = nch * _CH
    e_pad = epw * nw
    pad_e = e_pad - e

    src = jnp.concatenate(
        [edge_index[0], jnp.zeros((pad_e,), jnp.int32)])
    dst = jnp.concatenate(
        [edge_index[1], jnp.full((pad_e,), n, jnp.int32)])
    et = jnp.concatenate(
        [edge_type, jnp.zeros((pad_e,), jnp.int32)])

    x_p = jnp.zeros((np_rows, d), jnp.float32).at[:n].set(node_feature)
    nt_p = jnp.full((np_rows, 1), num_types, jnp.int32).at[:n, 0].set(node_type)
    zeros_nd = jnp.zeros((np_rows, d), jnp.float32)

    blk = np_rows // 8
    grid = (np_rows // blk,)
    full3 = lambda shape: pl.BlockSpec(shape, lambda i: (0, 0, 0))
    full2 = lambda shape: pl.BlockSpec(shape, lambda i: (0, 0))
    row_blk = lambda w: pl.BlockSpec((blk, w), lambda i: (i, 0))

    adapter = pl.pallas_call(
        _adapter_block,
        grid=grid,
        in_specs=[row_blk(d), row_blk(1),
                  full3((num_types, d, d)), full3((num_types, 1, d))],
        out_specs=row_blk(d),
        out_shape=jax.ShapeDtypeStruct((np_rows, d), jnp.float32),
    )
    h = adapter(x_p, nt_p, adapt_W, adapt_b[:, None, :])

    sc_counts = _make_sc_counts(np_rows, num_rel, nch, nc, ns)
    cnt_parts = sc_counts(et.reshape(nw, nch, _CH),
                          dst.reshape(nw, nch, _CH)).reshape(
                              nw, np_rows, num_rel)

    reduce_counts = pl.pallas_call(
        _reduce_counts_block,
        grid=grid,
        in_specs=[pl.BlockSpec((nw, blk, num_rel), lambda i: (0, i, 0))],
        out_specs=pl.BlockSpec((blk, num_rel), lambda i: (i, 0)),
        out_shape=jax.ShapeDtypeStruct((np_rows, num_rel), jnp.float32),
    )
    cmat = reduce_counts(cnt_parts)

    pairs = jnp.stack([src.reshape(nw * nch, _CH),
                       dst.reshape(nw * nch, _CH)], axis=1)
    nch0 = (2 * nch * 61 + 50) // 100
    nch1 = 2 * nch - nch0
    sc_seg = _make_sc_scatter(np_rows, d, nch0, nch1, rpt, nc, ns)

    layer = pl.pallas_call(
        _layer_block,
        grid=grid,
        in_specs=[row_blk(d), row_blk(d), row_blk(num_rel),
                  row_blk(d), full2((num_rel, d)), full2((d, d)),
                  full2((1, d))],
        out_specs=row_blk(d),
        out_shape=jax.ShapeDtypeStruct((np_rows, d), jnp.float32),
    )

    for l in range(num_layers):
        part = sc_seg(h, pairs, zeros_nd)
        h = layer(part[:np_rows], part[np_rows:], cmat, h,
                  rel_emb[l], conv_W[l], conv_b[l][None, :])

    return h[:n]


# speedup vs baseline: 1.1780x; 1.0310x over previous
import functools

import jax
import jax.numpy as jnp
from jax import lax
from jax.experimental import pallas as pl
from jax.experimental.pallas import tpu as pltpu
from jax.experimental.pallas import tpu_sc as plsc

_CH = 128
_CW = 16


def _adapter_block(x_ref, nt_ref, w_ref, b_ref, o_ref):
    x = x_ref[...]
    nt = nt_ref[...]
    acc = jnp.zeros_like(x)
    for t in range(w_ref.shape[0]):
        y = jnp.tanh(jnp.dot(x, w_ref[t], preferred_element_type=jnp.float32)
                     + b_ref[t])
        acc = jnp.where(nt == t, y, acc)
    o_ref[...] = acc


def _layer_block(s0_ref, s1_ref, c_ref, h_ref, rel_ref, w_ref, b_ref,
                 o_ref):
    cnt = c_ref[...]
    deg = jnp.maximum(jnp.sum(cnt, axis=1, keepdims=True), 1.0)
    base = (s0_ref[...] + s1_ref[...]
            + jnp.dot(cnt, rel_ref[...], preferred_element_type=jnp.float32))
    agg = base / deg
    o_ref[...] = (jnp.tanh(jnp.dot(agg, w_ref[...],
                                   preferred_element_type=jnp.float32)
                           + b_ref[...])
                  + h_ref[...])


def _make_sc_scatter(np_rows, width, nch0, nch1, rpt, nc, ns):
    mesh = plsc.VectorSubcoreMesh(core_axis_name="c", subcore_axis_name="s")

    @functools.partial(
        pl.kernel,
        out_type=jax.ShapeDtypeStruct((nc * np_rows, width), jnp.float32),
        mesh=mesh,
        scratch_types=[
            pltpu.VMEM((2, _CH), jnp.int32),
            pltpu.VMEM((_CH, width), jnp.float32),
            pltpu.SemaphoreType.DMA,
            pltpu.VMEM_SHARED((np_rows, width), jnp.float32),
        ],
    )
    def seg(table_hbm, pair_hbm, zeros_hbm, out_hbm, idx, rows, sem, acc):
        c = lax.axis_index("c")
        s = lax.axis_index("s")
        r0 = s * rpt
        nch_c = jnp.where(c == 0, nch0, nch1)
        base = jnp.where(c == 0, s * nch0, ns * nch0 + s * nch1)
        pltpu.sync_copy(zeros_hbm.at[pl.ds(r0, rpt)], acc.at[pl.ds(r0, rpt)])
        plsc.subcore_barrier()

        def body(j, carry):
            pltpu.sync_copy(pair_hbm.at[base + j], idx)
            pltpu.async_copy(table_hbm.at[idx.at[0]], rows, sem).wait()
            pltpu.sync_copy(rows, acc.at[idx.at[1]], add=True)
            return carry

        lax.fori_loop(0, nch_c, body, 0)
        plsc.subcore_barrier()
        pltpu.sync_copy(acc.at[pl.ds(r0, rpt)],
                        out_hbm.at[pl.ds(c * np_rows + r0, rpt)])

    return seg


def _make_sc_counts(np_rows, nr, nch, nc, ns):
    mesh = plsc.VectorSubcoreMesh(core_axis_name="c", subcore_axis_name="s")
    nw = nc * ns
    flat = np_rows * nr

    @functools.partial(
        pl.kernel,
        out_type=jax.ShapeDtypeStruct((nw, flat), jnp.float32),
        mesh=mesh,
        scratch_types=[
            pltpu.VMEM((nch, _CH), jnp.int32),
            pltpu.VMEM((nch, _CH), jnp.int32),
            pltpu.VMEM((flat,), jnp.float32),
        ],
        compiler_params=pltpu.CompilerParams(needs_layout_passes=False),
    )
    def counts(et_hbm, dst_hbm, out_hbm, ei, di, acc):
        c = lax.axis_index("c")
        s = lax.axis_index("s")
        w = c * ns + s
        zeros16 = jnp.zeros((16,), jnp.float32)
        ones = jnp.ones((16,), jnp.float32)
        pltpu.sync_copy(et_hbm.at[w], ei)
        pltpu.sync_copy(dst_hbm.at[w], di)

        def zbody(i, carry):
            acc[pl.ds(i * 16, 16)] = zeros16
            return carry

        lax.fori_loop(0, flat // 16, zbody, 0)

        def body(i, carry):
            for g in range(_CH // 16):
                idx = di[i, pl.ds(g * 16, 16)] * nr + ei[i, pl.ds(g * 16, 16)]
                plsc.addupdate_scatter(acc, [idx], ones)
            return carry

        lax.fori_loop(0, nch, body, 0)
        pltpu.sync_copy(acc, out_hbm.at[w])

    return counts


def _reduce_counts_block(parts_ref, o_ref):
    o_ref[...] = jnp.sum(parts_ref[...], axis=0)


def kernel(node_feature, node_type, edge_index, edge_type, node_position,
           adapt_W, adapt_b, conv_W, conv_b, rel_emb):
    n, d = node_feature.shape
    num_types = adapt_W.shape[0]
    num_layers, num_rel, _ = rel_emb.shape
    e = edge_index.shape[1]

    info = plsc.get_sparse_core_info()
    nc, ns = info.num_cores, info.num_subcores
    nw = nc * ns

    np_rows = ((n + 1 + 8 * ns - 1) // (8 * ns)) * (8 * ns)
    rpt = np_rows // ns
    nch = -(-e // (nw * _CH))
    epw = nch * _CH
    e_pad = epw * nw
    pad_e = e_pad - e

    src = jnp.concatenate(
        [edge_index[0], jnp.zeros((pad_e,), jnp.int32)])
    dst = jnp.concatenate(
        [edge_index[1], jnp.full((pad_e,), n, jnp.int32)])
    et = jnp.concatenate(
        [edge_type, jnp.zeros((pad_e,), jnp.int32)])

    x_p = jnp.zeros((np_rows, d), jnp.float32).at[:n].set(node_feature)
    nt_p = jnp.full((np_rows, 1), num_types, jnp.int32).at[:n, 0].set(node_type)
    zeros_nd = jnp.zeros((np_rows, d), jnp.float32)

    blk = np_rows // 8
    grid = (np_rows // blk,)
    full3 = lambda shape: pl.BlockSpec(shape, lambda i: (0, 0, 0))
    full2 = lambda shape: pl.BlockSpec(shape, lambda i: (0, 0))
    row_blk = lambda w: pl.BlockSpec((blk, w), lambda i: (i, 0))

    adapter = pl.pallas_call(
        _adapter_block,
        grid=grid,
        in_specs=[row_blk(d), row_blk(1),
                  full3((num_types, d, d)), full3((num_types, 1, d))],
        out_specs=row_blk(d),
        out_shape=jax.ShapeDtypeStruct((np_rows, d), jnp.float32),
    )
    h = adapter(x_p, nt_p, adapt_W, adapt_b[:, None, :])

    sc_counts = _make_sc_counts(np_rows, num_rel, nch, nc, ns)
    cnt_parts = sc_counts(et.reshape(nw, nch, _CH),
                          dst.reshape(nw, nch, _CH)).reshape(
                              nw, np_rows, num_rel)

    reduce_counts = pl.pallas_call(
        _reduce_counts_block,
        grid=grid,
        in_specs=[pl.BlockSpec((nw, blk, num_rel), lambda i: (0, i, 0))],
        out_specs=pl.BlockSpec((blk, num_rel), lambda i: (i, 0)),
        out_shape=jax.ShapeDtypeStruct((np_rows, num_rel), jnp.float32),
    )
    cmat = reduce_counts(cnt_parts)

    pairs = jnp.stack([src.reshape(nw * nch, _CH),
                       dst.reshape(nw * nch, _CH)], axis=1)
    nch0 = (2 * nch * 66 + 50) // 100
    nch1 = 2 * nch - nch0
    sc_seg = _make_sc_scatter(np_rows, d, nch0, nch1, rpt, nc, ns)

    layer = pl.pallas_call(
        _layer_block,
        grid=grid,
        in_specs=[row_blk(d), row_blk(d), row_blk(num_rel),
                  row_blk(d), full2((num_rel, d)), full2((d, d)),
                  full2((1, d))],
        out_specs=row_blk(d),
        out_shape=jax.ShapeDtypeStruct((np_rows, d), jnp.float32),
    )

    for l in range(num_layers):
        part = sc_seg(h, pairs, zeros_nd)
        h = layer(part[:np_rows], part[np_rows:], cmat, h,
                  rel_emb[l], conv_W[l], conv_b[l][None, :])

    return h[:n]


# speedup vs baseline: 1.2069x; 1.0246x over previous
import functools

import jax
import jax.numpy as jnp
from jax import lax
from jax.experimental import pallas as pl
from jax.experimental.pallas import tpu as pltpu
from jax.experimental.pallas import tpu_sc as plsc

_CH = 128
_CW = 16


def _adapter_block(x_ref, nt_ref, w_ref, b_ref, o_ref):
    x = x_ref[...]
    nt = nt_ref[...]
    acc = jnp.zeros_like(x)
    for t in range(w_ref.shape[0]):
        y = jnp.tanh(jnp.dot(x, w_ref[t], preferred_element_type=jnp.float32)
                     + b_ref[t])
        acc = jnp.where(nt == t, y, acc)
    o_ref[...] = acc


def _layer_block(s0_ref, s1_ref, c_ref, h_ref, rel_ref, w_ref, b_ref,
                 o_ref):
    cnt = c_ref[...]
    deg = jnp.maximum(jnp.sum(cnt, axis=1, keepdims=True), 1.0)
    base = (s0_ref[...] + s1_ref[...]
            + jnp.dot(cnt, rel_ref[...], preferred_element_type=jnp.float32))
    agg = base / deg
    o_ref[...] = (jnp.tanh(jnp.dot(agg, w_ref[...],
                                   preferred_element_type=jnp.float32)
                           + b_ref[...])
                  + h_ref[...])


def _make_sc_scatter(np_rows, width, nch0, nch1, rpt, nc, ns):
    mesh = plsc.VectorSubcoreMesh(core_axis_name="c", subcore_axis_name="s")

    @functools.partial(
        pl.kernel,
        out_type=jax.ShapeDtypeStruct((nc * np_rows, width), jnp.float32),
        mesh=mesh,
        scratch_types=[
            pltpu.VMEM((2, _CH), jnp.int32),
            pltpu.VMEM((_CH, width), jnp.float32),
            pltpu.SemaphoreType.DMA,
            pltpu.VMEM_SHARED((np_rows, width), jnp.float32),
        ],
    )
    def seg(table_hbm, pair_hbm, zeros_hbm, out_hbm, idx, rows, sem, acc):
        c = lax.axis_index("c")
        s = lax.axis_index("s")
        r0 = s * rpt
        nch_c = jnp.where(c == 0, nch0, nch1)
        base = jnp.where(c == 0, s * nch0, ns * nch0 + s * nch1)
        pltpu.sync_copy(zeros_hbm.at[pl.ds(r0, rpt)], acc.at[pl.ds(r0, rpt)])
        plsc.subcore_barrier()

        def body(j, carry):
            pltpu.sync_copy(pair_hbm.at[base + j], idx)
            pltpu.async_copy(table_hbm.at[idx.at[0]], rows, sem).wait()
            pltpu.sync_copy(rows, acc.at[idx.at[1]], add=True)
            return carry

        lax.fori_loop(0, nch_c, body, 0)
        plsc.subcore_barrier()
        pltpu.sync_copy(acc.at[pl.ds(r0, rpt)],
                        out_hbm.at[pl.ds(c * np_rows + r0, rpt)])

    return seg


def _make_sc_counts(np_rows, nr, nch, nc, ns):
    mesh = plsc.VectorSubcoreMesh(core_axis_name="c", subcore_axis_name="s")
    nw = nc * ns
    flat = np_rows * nr

    @functools.partial(
        pl.kernel,
        out_type=jax.ShapeDtypeStruct((nw, flat), jnp.float32),
        mesh=mesh,
        scratch_types=[
            pltpu.VMEM((nch, _CH), jnp.int32),
            pltpu.VMEM((nch, _CH), jnp.int32),
            pltpu.VMEM((flat,), jnp.float32),
        ],
        compiler_params=pltpu.CompilerParams(needs_layout_passes=False),
    )
    def counts(et_hbm, dst_hbm, out_hbm, ei, di, acc):
        c = lax.axis_index("c")
        s = lax.axis_index("s")
        w = c * ns + s
        zeros16 = jnp.zeros((16,), jnp.float32)
        ones = jnp.ones((16,), jnp.float32)
        pltpu.sync_copy(et_hbm.at[w], ei)
        pltpu.sync_copy(dst_hbm.at[w], di)

        def zbody(i, carry):
            acc[pl.ds(i * 16, 16)] = zeros16
            return carry

        lax.fori_loop(0, flat // 16, zbody, 0)

        def body(i, carry):
            for g in range(_CH // 16):
                idx = di[i, pl.ds(g * 16, 16)] * nr + ei[i, pl.ds(g * 16, 16)]
                plsc.addupdate_scatter(acc, [idx], ones)
            return carry

        lax.fori_loop(0, nch, body, 0)
        pltpu.sync_copy(acc, out_hbm.at[w])

    return counts


def _reduce_counts_block(parts_ref, o_ref):
    o_ref[...] = jnp.sum(parts_ref[...], axis=0)


def kernel(node_feature, node_type, edge_index, edge_type, node_position,
           adapt_W, adapt_b, conv_W, conv_b, rel_emb):
    n, d = node_feature.shape
    num_types = adapt_W.shape[0]
    num_layers, num_rel, _ = rel_emb.shape
    e = edge_index.shape[1]

    info = plsc.get_sparse_core_info()
    nc, ns = info.num_cores, info.num_subcores
    nw = nc * ns

    np_rows = ((n + 1 + 8 * ns - 1) // (8 * ns)) * (8 * ns)
    rpt = np_rows // ns
    nch = -(-e // (nw * _CH))
    epw = nch * _CH
    e_pad = epw * nw
    pad_e = e_pad - e

    src = jnp.concatenate(
        [edge_index[0], jnp.zeros((pad_e,), jnp.int32)])
    dst = jnp.concatenate(
        [edge_index[1], jnp.full((pad_e,), n, jnp.int32)])
    et = jnp.concatenate(
        [edge_type, jnp.zeros((pad_e,), jnp.int32)])

    x_p = jnp.zeros((np_rows, d), jnp.float32).at[:n].set(node_feature)
    nt_p = jnp.full((np_rows, 1), num_types, jnp.int32).at[:n, 0].set(node_type)
    zeros_nd = jnp.zeros((np_rows, d), jnp.float32)

    blk = np_rows // 8
    grid = (np_rows // blk,)
    full3 = lambda shape: pl.BlockSpec(shape, lambda i: (0, 0, 0))
    full2 = lambda shape: pl.BlockSpec(shape, lambda i: (0, 0))
    row_blk = lambda w: pl.BlockSpec((blk, w), lambda i: (i, 0))

    adapter = pl.pallas_call(
        _adapter_block,
        grid=grid,
        in_specs=[row_blk(d), row_blk(1),
                  full3((num_types, d, d)), full3((num_types, 1, d))],
        out_specs=row_blk(d),
        out_shape=jax.ShapeDtypeStruct((np_rows, d), jnp.float32),
    )
    h = adapter(x_p, nt_p, adapt_W, adapt_b[:, None, :])

    sc_counts = _make_sc_counts(np_rows, num_rel, nch, nc, ns)
    cnt_parts = sc_counts(et.reshape(nw, nch, _CH),
                          dst.reshape(nw, nch, _CH)).reshape(
                              nw, np_rows, num_rel)

    reduce_counts = pl.pallas_call(
        _reduce_counts_block,
        grid=grid,
        in_specs=[pl.BlockSpec((nw, blk, num_rel), lambda i: (0, i, 0))],
        out_specs=pl.BlockSpec((blk, num_rel), lambda i: (i, 0)),
        out_shape=jax.ShapeDtypeStruct((np_rows, num_rel), jnp.float32),
    )
    cmat = reduce_counts(cnt_parts)

    pairs = jnp.stack([src.reshape(nw * nch, _CH),
                       dst.reshape(nw * nch, _CH)], axis=1)
    nch0 = (2 * nch * 71 + 50) // 100
    nch1 = 2 * nch - nch0
    sc_seg = _make_sc_scatter(np_rows, d, nch0, nch1, rpt, nc, ns)

    layer = pl.pallas_call(
        _layer_block,
        grid=grid,
        in_specs=[row_blk(d), row_blk(d), row_blk(num_rel),
                  row_blk(d), full2((num_rel, d)), full2((d, d)),
                  full2((1, d))],
        out_specs=row_blk(d),
        out_shape=jax.ShapeDtypeStruct((np_rows, d), jnp.float32),
    )

    for l in range(num_layers):
        part = sc_seg(h, pairs, zeros_nd)
        h = layer(part[:np_rows], part[np_rows:], cmat, h,
                  rel_emb[l], conv_W[l], conv_b[l][None, :])

    return h[:n]
